# baseline (speedup 1.0000x reference)
import jax
import jax.numpy as jnp
from jax.experimental import pallas as pl

DIM = 64


def _dequant_body(x_ref, s_ref, o_ref):
  o_ref[...] = (x_ref[...] * s_ref[0:1, :]).astype(jnp.bfloat16)


def kernel(input, weight, weight_scaler):
  b, h = input.shape
  emb = jnp.take(weight, input.reshape(-1), axis=0)
  n = emb.shape[0]
  x = emb.reshape(n // 2, 2 * DIM)
  s = jnp.broadcast_to(
      jnp.tile(weight_scaler.astype(jnp.float32), 2).reshape(1, 2 * DIM),
      (8, 2 * DIM),
  )
  blk = 2048
  out = pl.pallas_call(
      _dequant_body,
      grid=(x.shape[0] // blk,),
      in_specs=[
          pl.BlockSpec((blk, 2 * DIM), lambda i: (i, 0)),
          pl.BlockSpec((8, 2 * DIM), lambda i: (0, 0)),
      ],
      out_specs=pl.BlockSpec((blk, 2 * DIM), lambda i: (i, 0)),
      out_shape=jax.ShapeDtypeStruct((x.shape[0], 2 * DIM), jnp.bfloat16),
  )(x, s)
  return out.reshape(b, h, DIM)
